# baseline (speedup 1.0000x reference)
import functools

import jax
import jax.numpy as jnp
from jax import lax
from jax.experimental import pallas as pl
from jax.experimental.pallas import tpu as pltpu
from jax.experimental.pallas import tpu_sc as plsc

_N_FIELDS = 26
_VOCAB = 100000
_EMB = 50
_BATCH = 16384
_N_CONT = 13
_N_EMB = _N_FIELDS * _EMB
_EPS = 1e-5

_R = _BATCH * _N_FIELDS
_NW = 32
_CHUNK = 128
_ROWS_PER_W = _R // _NW
_NCHUNK = _ROWS_PER_W // _CHUNK
_NPAIR = _NCHUNK // 2

_BB = 512


def _sc_gather(table, idx2d):
  mesh = plsc.VectorSubcoreMesh(core_axis_name="c", subcore_axis_name="s")

  @functools.partial(
      pl.kernel,
      mesh=mesh,
      compiler_params=pltpu.CompilerParams(use_tc_tiling_on_sc=False),
      out_type=jax.ShapeDtypeStruct((_R, _EMB), jnp.float32),
      scratch_types=[
          pltpu.VMEM((_NCHUNK, _CHUNK), jnp.int32),
          pltpu.VMEM((_CHUNK, _EMB), jnp.float32),
          pltpu.VMEM((_CHUNK, _EMB), jnp.float32),
          pltpu.SemaphoreType.DMA,
          pltpu.SemaphoreType.DMA,
      ],
  )
  def k(table_hbm, idx_hbm, out_hbm, idx_v, buf0, buf1, sem0, sem1):
    wid = lax.axis_index("s") * 2 + lax.axis_index("c")
    cbase = wid * _NCHUNK
    rbase = wid * _ROWS_PER_W
    pltpu.sync_copy(idx_hbm.at[pl.ds(cbase, _NCHUNK)], idx_v)

    def start(j, buf, sem):
      pltpu.async_copy(table_hbm.at[idx_v.at[j]], buf, sem)

    def wait(j, buf, sem):
      pltpu.make_async_copy(table_hbm.at[idx_v.at[j]], buf, sem).wait()

    def flush(j, buf):
      pltpu.sync_copy(buf, out_hbm.at[pl.ds(rbase + j * _CHUNK, _CHUNK)])

    start(0, buf0, sem0)

    def body(g, carry):
      j0 = 2 * g
      start(j0 + 1, buf1, sem1)
      wait(j0, buf0, sem0)
      flush(j0, buf0)

      @pl.when(g < _NPAIR - 1)
      def _():
        start(j0 + 2, buf0, sem0)

      wait(j0 + 1, buf1, sem1)
      flush(j0 + 1, buf1)
      return carry

    lax.fori_loop(0, _NPAIR, body, 0)

  return k(table, idx2d)


def _tc_mlp(xg, xc, w1e_t, w1c_t, b1r, w2_t, b2r, w3_t, b3r):
  def body(xg_ref, xc_ref, w1e_ref, w1c_ref, b1_ref, w2_ref, b2_ref,
           w3_ref, b3_ref, o_ref):
    z1 = jnp.dot(xg_ref[...], w1e_ref[...], preferred_element_type=jnp.float32)
    z1 = z1 + jnp.dot(xc_ref[...], w1c_ref[...],
                      preferred_element_type=jnp.float32)
    h1 = jnp.maximum(z1 + b1_ref[...], 0.0)
    z2 = jnp.dot(h1, w2_ref[...], preferred_element_type=jnp.float32)
    h2 = jnp.maximum(z2 + b2_ref[...], 0.0)
    o_ref[...] = (jnp.dot(h2, w3_ref[...], preferred_element_type=jnp.float32)
                  + b3_ref[...])

  return pl.pallas_call(
      body,
      grid=(_BATCH // _BB,),
      in_specs=[
          pl.BlockSpec((_BB, _N_EMB), lambda i: (i, 0)),
          pl.BlockSpec((_BB, _N_CONT), lambda i: (i, 0)),
          pl.BlockSpec((_N_EMB, 16), lambda i: (0, 0)),
          pl.BlockSpec((_N_CONT, 16), lambda i: (0, 0)),
          pl.BlockSpec((1, 16), lambda i: (0, 0)),
          pl.BlockSpec((16, 32), lambda i: (0, 0)),
          pl.BlockSpec((1, 32), lambda i: (0, 0)),
          pl.BlockSpec((32, 1), lambda i: (0, 0)),
          pl.BlockSpec((1, 1), lambda i: (0, 0)),
      ],
      out_specs=pl.BlockSpec((_BB, 1), lambda i: (i, 0)),
      out_shape=jax.ShapeDtypeStruct((_BATCH, 1), jnp.float32),
  )(xg, xc, w1e_t, w1c_t, b1r, w2_t, b2r, w3_t, b3r)


def kernel(x_cat, x_cont, emb_tables, W1, b1, W2, b2, W3, b3,
           bn1_g, bn1_b, bn2_g, bn2_b, bn3_g, bn3_b):
  s = (1.0 / jnp.sqrt(1.0 + _EPS)).astype(jnp.float32)
  table = emb_tables.reshape(_N_FIELDS * _VOCAB, _EMB)
  offs = (jnp.arange(_N_FIELDS, dtype=jnp.int32) * _VOCAB)[None, :]
  gidx = (x_cat + offs).reshape(_R // _CHUNK, _CHUNK)

  xg = _sc_gather(table, gidx).reshape(_BATCH, _N_EMB)

  W1e = W1[:, :_N_EMB]
  W1c_orig = W1[:, _N_EMB:]
  W1c = W1c_orig * (s * bn1_g)[None, :]
  b1_eff = b1 + W1c_orig @ bn1_b
  W2e = W2 * (s * bn2_g)[None, :]
  b2_eff = b2 + W2 @ bn2_b
  W3e = W3 * (s * bn3_g)[None, :]
  b3_eff = b3 + W3 @ bn3_b

  return _tc_mlp(xg, x_cont, W1e.T, W1c.T, b1_eff.reshape(1, 16),
                 W2e.T, b2_eff.reshape(1, 32), W3e.T, b3_eff.reshape(1, 1))


# speedup vs baseline: 1.6817x; 1.6817x over previous
import functools

import jax
import jax.numpy as jnp
from jax import lax
from jax.experimental import pallas as pl
from jax.experimental.pallas import tpu as pltpu
from jax.experimental.pallas import tpu_sc as plsc

_N_FIELDS = 26
_VOCAB = 100000
_EMB = 50
_BATCH = 16384
_N_CONT = 13
_N_EMB = _N_FIELDS * _EMB
_EPS = 1e-5

_H1 = 16
_PACK = 128 // _H1
_NPROW = _N_FIELDS * _VOCAB // _PACK

_QA = 125
_QB = 100
_BA = 5
_G1F = _QA // _BA

_NW = 32
_SPW = _BATCH // _NW
_SPC = 4
_CPW = _SPW // _SPC
_RPC = _SPC * _N_FIELDS
_NPAIR = _CPW // 2

_BB = 2048


def _tc_project(x5, wbig):
  def body(x_ref, w_ref, o_ref):
    x = x_ref[0]
    xw = x.reshape(_BA * _QB, _PACK, _EMB)
    acc = jnp.dot(xw[:, 0, :], w_ref[0, 0],
                  preferred_element_type=jnp.float32)
    for s in range(1, _PACK):
      acc = acc + jnp.dot(xw[:, s, :], w_ref[0, s],
                          preferred_element_type=jnp.float32)
    o_ref[0] = acc.reshape(_BA, _QB, 128)

  return pl.pallas_call(
      body,
      grid=(_N_FIELDS, _G1F),
      in_specs=[
          pl.BlockSpec((1, _BA, _QB, _PACK, _EMB),
                       lambda f, i: (f, i, 0, 0, 0)),
          pl.BlockSpec((1, _PACK, _EMB, 128), lambda f, i: (f, 0, 0, 0)),
      ],
      out_specs=pl.BlockSpec((1, _BA, _QB, 128), lambda f, i: (f, i, 0, 0)),
      out_shape=jax.ShapeDtypeStruct((_N_FIELDS, _QA, _QB, 128), jnp.float32),
  )(x5, wbig)


def _sc_bag(p, prow, poff):
  mesh = plsc.VectorSubcoreMesh(core_axis_name="c", subcore_axis_name="s")

  @functools.partial(
      pl.kernel,
      mesh=mesh,
      compiler_params=pltpu.CompilerParams(use_tc_tiling_on_sc=False),
      out_type=jax.ShapeDtypeStruct((_BATCH * _H1 // 128, 128), jnp.float32),
      scratch_types=[
          pltpu.VMEM((_CPW, 128), jnp.int32),
          pltpu.VMEM((_CPW, 128), jnp.int32),
          pltpu.VMEM((_RPC, 128), jnp.float32),
          pltpu.VMEM((_RPC, 128), jnp.float32),
          pltpu.VMEM((_SPW * _H1 // 128, 128), jnp.float32),
          pltpu.SemaphoreType.DMA,
          pltpu.SemaphoreType.DMA,
      ],
  )
  def k(p_hbm, prow_hbm, poff_hbm, out_hbm, prow_v, poff_v, buf0, buf1,
        acc_v, sem0, sem1):
    wid = lax.axis_index("s") * 2 + lax.axis_index("c")
    pltpu.sync_copy(prow_hbm.at[wid], prow_v)
    pltpu.sync_copy(poff_hbm.at[wid], poff_v)

    def start(c, buf, sem):
      pltpu.async_copy(p_hbm.at[prow_v.at[c, pl.ds(0, _RPC)]], buf, sem)

    def wait(c, buf, sem):
      pltpu.make_async_copy(p_hbm.at[prow_v.at[c, pl.ds(0, _RPC)]], buf,
                            sem).wait()

    def process(c, buf):
      for j in range(_SPC):
        acc = jnp.zeros((_H1,), jnp.float32)
        offvecs = {}
        for f in range(_N_FIELDS):
          e = j * _N_FIELDS + f
          g = e // 16
          if g not in offvecs:
            offvecs[g] = poff_v[c, pl.ds(16 * g, 16)]
          off = offvecs[g][e % 16]
          acc = acc + buf[e, pl.ds(off, _H1)]
        s_loc = c * _SPC + j
        acc_v[s_loc >> 3, pl.ds((s_loc & 7) * _H1, _H1)] = acc

    start(0, buf0, sem0)

    def body(g, carry):
      c0 = 2 * g
      start(c0 + 1, buf1, sem1)
      wait(c0, buf0, sem0)
      process(c0, buf0)

      @pl.when(g < _NPAIR - 1)
      def _():
        start(c0 + 2, buf0, sem0)

      wait(c0 + 1, buf1, sem1)
      process(c0 + 1, buf1)
      return carry

    lax.fori_loop(0, _NPAIR, body, 0)
    pltpu.sync_copy(acc_v, out_hbm.at[pl.ds(wid * (_SPW * _H1 // 128),
                                            _SPW * _H1 // 128)])

  return k(p, prow, poff)


def _tc_mlp(e1, xc, w1c_t, b1r, w2_t, b2r, w3_t, b3r):
  def body(e1_ref, xc_ref, w1c_ref, b1_ref, w2_ref, b2_ref, w3_ref, b3_ref,
           o_ref):
    z1 = e1_ref[...] + jnp.dot(xc_ref[...], w1c_ref[...],
                               preferred_element_type=jnp.float32)
    h1 = jnp.maximum(z1 + b1_ref[...], 0.0)
    z2 = jnp.dot(h1, w2_ref[...], preferred_element_type=jnp.float32)
    h2 = jnp.maximum(z2 + b2_ref[...], 0.0)
    o_ref[...] = (jnp.dot(h2, w3_ref[...], preferred_element_type=jnp.float32)
                  + b3_ref[...])

  return pl.pallas_call(
      body,
      grid=(_BATCH // _BB,),
      in_specs=[
          pl.BlockSpec((_BB, _H1), lambda i: (i, 0)),
          pl.BlockSpec((_BB, _N_CONT), lambda i: (i, 0)),
          pl.BlockSpec((_N_CONT, _H1), lambda i: (0, 0)),
          pl.BlockSpec((1, _H1), lambda i: (0, 0)),
          pl.BlockSpec((_H1, 32), lambda i: (0, 0)),
          pl.BlockSpec((1, 32), lambda i: (0, 0)),
          pl.BlockSpec((32, 1), lambda i: (0, 0)),
          pl.BlockSpec((1, 1), lambda i: (0, 0)),
      ],
      out_specs=pl.BlockSpec((_BB, 1), lambda i: (i, 0)),
      out_shape=jax.ShapeDtypeStruct((_BATCH, 1), jnp.float32),
  )(e1, xc, w1c_t, b1r, w2_t, b2r, w3_t, b3r)


def kernel(x_cat, x_cont, emb_tables, W1, b1, W2, b2, W3, b3,
           bn1_g, bn1_b, bn2_g, bn2_b, bn3_g, bn3_b):
  s = (1.0 / jnp.sqrt(1.0 + _EPS)).astype(jnp.float32)

  w1f = W1[:, :_N_EMB].reshape(_H1, _N_FIELDS, _EMB).transpose(1, 2, 0)
  eye = jnp.eye(_PACK, dtype=jnp.float32)
  wbig = (w1f[:, None, :, None, :] * eye[None, :, None, :, None]).reshape(
      _N_FIELDS, _PACK, _EMB, 128)

  x5 = emb_tables.reshape(_N_FIELDS, _QA, _QB, _PACK, _EMB)
  p = _tc_project(x5, wbig).reshape(_NPROW, 128)

  offs = (jnp.arange(_N_FIELDS, dtype=jnp.int32) * _VOCAB)[None, :]
  r_flat = (x_cat + offs).reshape(_NW, _CPW, _RPC)
  pad = ((0, 0), (0, 0), (0, 128 - _RPC))
  prow = jnp.pad(r_flat >> 3, pad)
  poff = jnp.pad((r_flat & 7) << 4, pad)

  e1 = _sc_bag(p, prow, poff).reshape(_BATCH, _H1)

  W1c_orig = W1[:, _N_EMB:]
  W1c = W1c_orig * (s * bn1_g)[None, :]
  b1_eff = b1 + W1c_orig @ bn1_b
  W2e = W2 * (s * bn2_g)[None, :]
  b2_eff = b2 + W2 @ bn2_b
  W3e = W3 * (s * bn3_g)[None, :]
  b3_eff = b3 + W3 @ bn3_b

  return _tc_mlp(e1, x_cont, W1c.T, b1_eff.reshape(1, _H1),
                 W2e.T, b2_eff.reshape(1, 32), W3e.T, b3_eff.reshape(1, 1))


# speedup vs baseline: 3.4985x; 2.0803x over previous
import functools

import jax
import jax.numpy as jnp
from jax import lax
from jax.experimental import pallas as pl
from jax.experimental.pallas import tpu as pltpu
from jax.experimental.pallas import tpu_sc as plsc

_N_FIELDS = 26
_VOCAB = 100000
_EMB = 50
_BATCH = 16384
_N_CONT = 13
_N_EMB = _N_FIELDS * _EMB
_EPS = 1e-5

_H1 = 16
_PACK = 128 // _H1
_NPROW = _N_FIELDS * _VOCAB // _PACK

_VB = 4096
_G1F = -(-_VOCAB // _VB)
_FROWS = _G1F * _VB // _PACK
_NPROWP = _N_FIELDS * _FROWS

_NW = 32
_SPW = _BATCH // _NW
_SPC = 4
_CPW = _SPW // _SPC
_RPC = _SPC * _N_FIELDS
_NPAIR = _CPW // 2

_BB = 2048


def _tc_project(t_t, w1f):
  def body(x_ref, w_ref, o_ref):
    q = _VB // _PACK
    parts = []
    for t in range(_PACK):
      parts.append(lax.dot_general(x_ref[0, :, pl.ds(t * q, q)], w_ref[0],
                                   (((0,), (0,)), ((), ())),
                                   preferred_element_type=jnp.float32))
    o_ref[0] = jnp.concatenate(parts, axis=1)

  return pl.pallas_call(
      body,
      grid=(_N_FIELDS, _G1F),
      compiler_params=pltpu.CompilerParams(fuse_transposed_lhs_in_matmul=True),
      in_specs=[
          pl.BlockSpec((1, _EMB, _VB), lambda f, i: (f, 0, i)),
          pl.BlockSpec((1, _EMB, _H1), lambda f, i: (f, 0, 0)),
      ],
      out_specs=pl.BlockSpec((1, _VB // _PACK, 128), lambda f, i: (f, i, 0)),
      out_shape=jax.ShapeDtypeStruct((_N_FIELDS, _FROWS, 128), jnp.float32),
  )(t_t, w1f)


def _sc_bag(p, prow, poff):
  mesh = plsc.VectorSubcoreMesh(core_axis_name="c", subcore_axis_name="s")

  @functools.partial(
      pl.kernel,
      mesh=mesh,
      compiler_params=pltpu.CompilerParams(use_tc_tiling_on_sc=False),
      out_type=jax.ShapeDtypeStruct((_BATCH * _H1 // 128, 128), jnp.float32),
      scratch_types=[
          pltpu.VMEM((_CPW, 128), jnp.int32),
          pltpu.VMEM((_CPW, 128), jnp.int32),
          pltpu.VMEM((_RPC, 128), jnp.float32),
          pltpu.VMEM((_RPC, 128), jnp.float32),
          pltpu.VMEM((_SPW * _H1 // 128, 128), jnp.float32),
          pltpu.SemaphoreType.DMA,
          pltpu.SemaphoreType.DMA,
      ],
  )
  def k(p_hbm, prow_hbm, poff_hbm, out_hbm, prow_v, poff_v, buf0, buf1,
        acc_v, sem0, sem1):
    wid = lax.axis_index("s") * 2 + lax.axis_index("c")
    pltpu.sync_copy(prow_hbm.at[wid], prow_v)
    pltpu.sync_copy(poff_hbm.at[wid], poff_v)

    def start(c, buf, sem):
      pltpu.async_copy(p_hbm.at[prow_v.at[c, pl.ds(0, _RPC)]], buf, sem)

    def wait(c, buf, sem):
      pltpu.make_async_copy(p_hbm.at[prow_v.at[c, pl.ds(0, _RPC)]], buf,
                            sem).wait()

    def process(c, buf):
      for j in range(_SPC):
        acc = jnp.zeros((_H1,), jnp.float32)
        offvecs = {}
        for f in range(_N_FIELDS):
          e = j * _N_FIELDS + f
          g = e // 16
          if g not in offvecs:
            offvecs[g] = poff_v[c, pl.ds(16 * g, 16)]
          off = offvecs[g][e % 16]
          acc = acc + buf[e, pl.ds(off, _H1)]
        s_loc = c * _SPC + j
        acc_v[s_loc >> 3, pl.ds((s_loc & 7) * _H1, _H1)] = acc

    start(0, buf0, sem0)

    def body(g, carry):
      c0 = 2 * g
      start(c0 + 1, buf1, sem1)
      wait(c0, buf0, sem0)
      process(c0, buf0)

      @pl.when(g < _NPAIR - 1)
      def _():
        start(c0 + 2, buf0, sem0)

      wait(c0 + 1, buf1, sem1)
      process(c0 + 1, buf1)
      return carry

    lax.fori_loop(0, _NPAIR, body, 0)
    pltpu.sync_copy(acc_v, out_hbm.at[pl.ds(wid * (_SPW * _H1 // 128),
                                            _SPW * _H1 // 128)])

  return k(p, prow, poff)


def _tc_mlp(e1, xc, w1c_t, b1r, w2_t, b2r, w3_t, b3r):
  def body(e1_ref, xc_ref, w1c_ref, b1_ref, w2_ref, b2_ref, w3_ref, b3_ref,
           o_ref):
    z1 = e1_ref[...] + jnp.dot(xc_ref[...], w1c_ref[...],
                               preferred_element_type=jnp.float32)
    h1 = jnp.maximum(z1 + b1_ref[...], 0.0)
    z2 = jnp.dot(h1, w2_ref[...], preferred_element_type=jnp.float32)
    h2 = jnp.maximum(z2 + b2_ref[...], 0.0)
    o_ref[...] = (jnp.dot(h2, w3_ref[...], preferred_element_type=jnp.float32)
                  + b3_ref[...])

  return pl.pallas_call(
      body,
      grid=(_BATCH // _BB,),
      in_specs=[
          pl.BlockSpec((_BB, _H1), lambda i: (i, 0)),
          pl.BlockSpec((_BB, _N_CONT), lambda i: (i, 0)),
          pl.BlockSpec((_N_CONT, _H1), lambda i: (0, 0)),
          pl.BlockSpec((1, _H1), lambda i: (0, 0)),
          pl.BlockSpec((_H1, 32), lambda i: (0, 0)),
          pl.BlockSpec((1, 32), lambda i: (0, 0)),
          pl.BlockSpec((32, 1), lambda i: (0, 0)),
          pl.BlockSpec((1, 1), lambda i: (0, 0)),
      ],
      out_specs=pl.BlockSpec((_BB, 1), lambda i: (i, 0)),
      out_shape=jax.ShapeDtypeStruct((_BATCH, 1), jnp.float32),
  )(e1, xc, w1c_t, b1r, w2_t, b2r, w3_t, b3r)


def kernel(x_cat, x_cont, emb_tables, W1, b1, W2, b2, W3, b3,
           bn1_g, bn1_b, bn2_g, bn2_b, bn3_g, bn3_b):
  s = (1.0 / jnp.sqrt(1.0 + _EPS)).astype(jnp.float32)

  w1f = W1[:, :_N_EMB].reshape(_H1, _N_FIELDS, _EMB).transpose(1, 2, 0)

  t_t = emb_tables.transpose(0, 2, 1)
  p = _tc_project(t_t, w1f).reshape(_NPROWP, 128)

  offs = (jnp.arange(_N_FIELDS, dtype=jnp.int32) * _FROWS)[None, :]
  q = _VB // _PACK
  prow_full = (x_cat // _VB) * q + (x_cat % q) + offs
  poff_full = ((x_cat % _VB) // q) * _H1
  pad = ((0, 0), (0, 0), (0, 128 - _RPC))
  prow = jnp.pad(prow_full.reshape(_NW, _CPW, _RPC), pad)
  poff = jnp.pad(poff_full.reshape(_NW, _CPW, _RPC), pad)

  e1 = _sc_bag(p, prow, poff).reshape(_BATCH, _H1)

  W1c_orig = W1[:, _N_EMB:]
  W1c = W1c_orig * (s * bn1_g)[None, :]
  b1_eff = b1 + W1c_orig @ bn1_b
  W2e = W2 * (s * bn2_g)[None, :]
  b2_eff = b2 + W2 @ bn2_b
  W3e = W3 * (s * bn3_g)[None, :]
  b3_eff = b3 + W3 @ bn3_b

  return _tc_mlp(e1, x_cont, W1c.T, b1_eff.reshape(1, _H1),
                 W2e.T, b2_eff.reshape(1, 32), W3e.T, b3_eff.reshape(1, 1))


# speedup vs baseline: 4.7820x; 1.3669x over previous
import functools

import jax
import jax.numpy as jnp
from jax import lax
from jax.experimental import pallas as pl
from jax.experimental.pallas import tpu as pltpu
from jax.experimental.pallas import tpu_sc as plsc

_N_FIELDS = 26
_VOCAB = 100000
_EMB = 50
_BATCH = 16384
_N_CONT = 13
_N_EMB = _N_FIELDS * _EMB
_EPS = 1e-5

_H1 = 16
_PACK = 128 // _H1
_NPROW = _N_FIELDS * _VOCAB // _PACK

_VB = 8192
_G1F = -(-_VOCAB // _VB)
_FROWS = _G1F * _VB // _PACK
_NPROWP = _N_FIELDS * _FROWS

_NW = 32
_SPW = _BATCH // _NW
_SPC = 4
_CPW = _SPW // _SPC
_RPC = _SPC * _N_FIELDS
_NPAIR = _CPW // 2

_BB = 2048


def _tc_project(t_t, w1ft, e8):
  def body(x_ref, w_ref, e_ref, o_ref):
    q = _VB // _PACK
    yt = lax.dot_general(w_ref[0], x_ref[0], (((1,), (0,)), ((), ())),
                         preferred_element_type=jnp.float32)
    acc = lax.dot_general(lax.slice(yt, (0, 0), (_H1, q)), e_ref[0],
                          (((0,), (0,)), ((), ())),
                          preferred_element_type=jnp.float32)
    for t in range(1, _PACK):
      acc = acc + lax.dot_general(
          lax.slice(yt, (0, t * q), (_H1, (t + 1) * q)), e_ref[t],
          (((0,), (0,)), ((), ())), preferred_element_type=jnp.float32)
    o_ref[0] = acc

  return pl.pallas_call(
      body,
      grid=(_N_FIELDS, _G1F),
      compiler_params=pltpu.CompilerParams(fuse_transposed_lhs_in_matmul=True),
      in_specs=[
          pl.BlockSpec((1, _EMB, _VB), lambda f, i: (f, 0, i)),
          pl.BlockSpec((1, _H1, _EMB), lambda f, i: (f, 0, 0)),
          pl.BlockSpec((_PACK, _H1, 128), lambda f, i: (0, 0, 0)),
      ],
      out_specs=pl.BlockSpec((1, _VB // _PACK, 128), lambda f, i: (f, i, 0)),
      out_shape=jax.ShapeDtypeStruct((_N_FIELDS, _FROWS, 128), jnp.float32),
  )(t_t, w1ft, e8)


def _sc_bag(p, prow, poff):
  mesh = plsc.VectorSubcoreMesh(core_axis_name="c", subcore_axis_name="s")

  @functools.partial(
      pl.kernel,
      mesh=mesh,
      compiler_params=pltpu.CompilerParams(use_tc_tiling_on_sc=False),
      out_type=jax.ShapeDtypeStruct((_BATCH * _H1 // 128, 128), jnp.float32),
      scratch_types=[
          pltpu.VMEM((_CPW, 128), jnp.int32),
          pltpu.VMEM((_CPW, 128), jnp.int32),
          pltpu.VMEM((_RPC, 128), jnp.float32),
          pltpu.VMEM((_RPC, 128), jnp.float32),
          pltpu.VMEM((_SPW * _H1 // 128, 128), jnp.float32),
          pltpu.SemaphoreType.DMA,
          pltpu.SemaphoreType.DMA,
      ],
  )
  def k(p_hbm, prow_hbm, poff_hbm, out_hbm, prow_v, poff_v, buf0, buf1,
        acc_v, sem0, sem1):
    wid = lax.axis_index("s") * 2 + lax.axis_index("c")
    pltpu.sync_copy(prow_hbm.at[wid], prow_v)
    pltpu.sync_copy(poff_hbm.at[wid], poff_v)

    def start(c, buf, sem):
      pltpu.async_copy(p_hbm.at[prow_v.at[c, pl.ds(0, _RPC)]], buf, sem)

    def wait(c, buf, sem):
      pltpu.make_async_copy(p_hbm.at[prow_v.at[c, pl.ds(0, _RPC)]], buf,
                            sem).wait()

    def process(c, buf):
      for j in range(_SPC):
        acc = jnp.zeros((_H1,), jnp.float32)
        offvecs = {}
        for f in range(_N_FIELDS):
          e = j * _N_FIELDS + f
          g = e // 16
          if g not in offvecs:
            offvecs[g] = poff_v[c, pl.ds(16 * g, 16)]
          off = offvecs[g][e % 16]
          acc = acc + buf[e, pl.ds(off, _H1)]
        s_loc = c * _SPC + j
        acc_v[s_loc >> 3, pl.ds((s_loc & 7) * _H1, _H1)] = acc

    start(0, buf0, sem0)

    def body(g, carry):
      c0 = 2 * g
      start(c0 + 1, buf1, sem1)
      wait(c0, buf0, sem0)
      process(c0, buf0)

      @pl.when(g < _NPAIR - 1)
      def _():
        start(c0 + 2, buf0, sem0)

      wait(c0 + 1, buf1, sem1)
      process(c0 + 1, buf1)
      return carry

    lax.fori_loop(0, _NPAIR, body, 0)
    pltpu.sync_copy(acc_v, out_hbm.at[pl.ds(wid * (_SPW * _H1 // 128),
                                            _SPW * _H1 // 128)])

  return k(p, prow, poff)


def _tc_mlp(e1, xc, w1c_t, b1r, w2_t, b2r, w3_t, b3r):
  def body(e1_ref, xc_ref, w1c_ref, b1_ref, w2_ref, b2_ref, w3_ref, b3_ref,
           o_ref):
    z1 = e1_ref[...] + jnp.dot(xc_ref[...], w1c_ref[...],
                               preferred_element_type=jnp.float32)
    h1 = jnp.maximum(z1 + b1_ref[...], 0.0)
    z2 = jnp.dot(h1, w2_ref[...], preferred_element_type=jnp.float32)
    h2 = jnp.maximum(z2 + b2_ref[...], 0.0)
    o_ref[...] = (jnp.dot(h2, w3_ref[...], preferred_element_type=jnp.float32)
                  + b3_ref[...])

  return pl.pallas_call(
      body,
      grid=(_BATCH // _BB,),
      in_specs=[
          pl.BlockSpec((_BB, _H1), lambda i: (i, 0)),
          pl.BlockSpec((_BB, _N_CONT), lambda i: (i, 0)),
          pl.BlockSpec((_N_CONT, _H1), lambda i: (0, 0)),
          pl.BlockSpec((1, _H1), lambda i: (0, 0)),
          pl.BlockSpec((_H1, 32), lambda i: (0, 0)),
          pl.BlockSpec((1, 32), lambda i: (0, 0)),
          pl.BlockSpec((32, 1), lambda i: (0, 0)),
          pl.BlockSpec((1, 1), lambda i: (0, 0)),
      ],
      out_specs=pl.BlockSpec((_BB, 1), lambda i: (i, 0)),
      out_shape=jax.ShapeDtypeStruct((_BATCH, 1), jnp.float32),
  )(e1, xc, w1c_t, b1r, w2_t, b2r, w3_t, b3r)


def kernel(x_cat, x_cont, emb_tables, W1, b1, W2, b2, W3, b3,
           bn1_g, bn1_b, bn2_g, bn2_b, bn3_g, bn3_b):
  s = (1.0 / jnp.sqrt(1.0 + _EPS)).astype(jnp.float32)

  w1ft = W1[:, :_N_EMB].reshape(_H1, _N_FIELDS, _EMB).transpose(1, 0, 2)
  e8 = (jnp.arange(128)[None, None, :] ==
        (_H1 * jnp.arange(_PACK)[:, None, None] +
         jnp.arange(_H1)[None, :, None])).astype(jnp.float32)

  t_t = emb_tables.transpose(0, 2, 1)
  p = _tc_project(t_t, w1ft, e8).reshape(_NPROWP, 128)

  offs = (jnp.arange(_N_FIELDS, dtype=jnp.int32) * _FROWS)[None, :]
  q = _VB // _PACK
  prow_full = (x_cat // _VB) * q + (x_cat % q) + offs
  poff_full = ((x_cat % _VB) // q) * _H1
  pad = ((0, 0), (0, 0), (0, 128 - _RPC))
  prow = jnp.pad(prow_full.reshape(_NW, _CPW, _RPC), pad)
  poff = jnp.pad(poff_full.reshape(_NW, _CPW, _RPC), pad)

  e1 = _sc_bag(p, prow, poff).reshape(_BATCH, _H1)

  W1c_orig = W1[:, _N_EMB:]
  W1c = W1c_orig * (s * bn1_g)[None, :]
  b1_eff = b1 + W1c_orig @ bn1_b
  W2e = W2 * (s * bn2_g)[None, :]
  b2_eff = b2 + W2 @ bn2_b
  W3e = W3 * (s * bn3_g)[None, :]
  b3_eff = b3 + W3 @ bn3_b

  return _tc_mlp(e1, x_cont, W1c.T, b1_eff.reshape(1, _H1),
                 W2e.T, b2_eff.reshape(1, 32), W3e.T, b3_eff.reshape(1, 1))
